# Initial kernel scaffold; baseline (speedup 1.0000x reference)
import jax
import jax.numpy as jnp
from jax.experimental import pallas as pl


def kernel(index, targets, embedding_table):
    raise NotImplementedError("write your pallas kernel here")


# speedup vs baseline: 1.4080x; 1.4080x over previous
import functools

import jax
import jax.numpy as jnp
from jax import lax
from jax.experimental import pallas as pl
from jax.experimental.pallas import tpu as pltpu
from jax.experimental.pallas import tpu_sc as plsc

_INFO = plsc.get_sparse_core_info()
_NC = _INFO.num_cores
_NS = _INFO.num_subcores
_NW = _NC * _NS

CHARSET = 1000
B, L = 4096, 20
_N = B * L
_PER_W = _N // _NW
_C = 64
_NCHUNK = _PER_W // _C


def _gather_body(table_hbm, idx_hbm, out_hbm, idx_v, rows_v, sem):
    wid = lax.axis_index("s") * _NC + lax.axis_index("c")
    base = wid * _PER_W
    pltpu.sync_copy(idx_hbm.at[wid], idx_v)

    def chunk(j, carry):
        pltpu.async_copy(table_hbm.at[idx_v.at[j]], rows_v, sem).wait()
        pltpu.sync_copy(rows_v, out_hbm.at[pl.ds(base + j * _C, _C)])
        return carry

    lax.fori_loop(0, _NCHUNK, chunk, 0)


@jax.jit
def _run(table, idx3):
    mesh = plsc.VectorSubcoreMesh(core_axis_name="c", subcore_axis_name="s")
    f = pl.kernel(
        _gather_body,
        out_type=jax.ShapeDtypeStruct((_N, CHARSET), jnp.float32),
        mesh=mesh,
        scratch_types=[
            pltpu.VMEM((_NCHUNK, _C), jnp.int32),
            pltpu.VMEM((_C, CHARSET), jnp.float32),
            pltpu.SemaphoreType.DMA,
        ],
        compiler_params=pltpu.CompilerParams(use_tc_tiling_on_sc=False),
    )
    return f(table, idx3)


def kernel(index, targets, embedding_table):
    idx3 = index.astype(jnp.int32).reshape(_NW, _NCHUNK, _C)
    out = _run(embedding_table, idx3)
    return out.reshape(B, L, CHARSET)


# speedup vs baseline: 1.4265x; 1.0132x over previous
import jax
import jax.numpy as jnp
from jax import lax
from jax.experimental import pallas as pl
from jax.experimental.pallas import tpu as pltpu
from jax.experimental.pallas import tpu_sc as plsc

_INFO = plsc.get_sparse_core_info()
_NC = _INFO.num_cores
_NS = _INFO.num_subcores
_NW = _NC * _NS

CHARSET = 1000
B, L = 4096, 20
_N = B * L
_PER_W = _N // _NW
_C = 32
_NCHUNK = _PER_W // _C
_NBUF = 4
_T = _NCHUNK // _NBUF


def _gather_body(table_hbm, idx_hbm, out_hbm, idx_v,
                 r0, r1, r2, r3, g0, g1, g2, g3, w0, w1, w2, w3):
    bufs = [r0, r1, r2, r3]
    gs = [g0, g1, g2, g3]
    ws = [w0, w1, w2, w3]
    wid = lax.axis_index("s") * _NC + lax.axis_index("c")
    base = wid * _PER_W
    pltpu.sync_copy(idx_hbm.at[wid], idx_v)

    def start_gather(j, b):
        pltpu.async_copy(table_hbm.at[idx_v.at[j]], bufs[b], gs[b])

    def wait_gather(j, b):
        pltpu.make_async_copy(table_hbm.at[idx_v.at[j]], bufs[b], gs[b]).wait()

    def start_write(j, b):
        pltpu.async_copy(bufs[b], out_hbm.at[pl.ds(base + j * _C, _C)], ws[b])

    def wait_write(j, b):
        pltpu.make_async_copy(
            bufs[b], out_hbm.at[pl.ds(base + j * _C, _C)], ws[b]).wait()

    for b in range(_NBUF):
        start_gather(b, b)

    def outer(g, carry):
        jj = g * _NBUF
        for b in range(_NBUF):
            wait_gather(jj + b, b)
            start_write(jj + b, b)
        for b in range(_NBUF):
            wait_write(jj + b, b)
            start_gather(jj + _NBUF + b, b)
        return carry

    lax.fori_loop(0, _T - 1, outer, 0)

    jj = (_T - 1) * _NBUF
    for b in range(_NBUF):
        wait_gather(jj + b, b)
        start_write(jj + b, b)
    for b in range(_NBUF):
        wait_write(jj + b, b)


@jax.jit
def _run(table, idx3):
    mesh = plsc.VectorSubcoreMesh(core_axis_name="c", subcore_axis_name="s")
    f = pl.kernel(
        _gather_body,
        out_type=jax.ShapeDtypeStruct((_N, CHARSET), jnp.float32),
        mesh=mesh,
        scratch_types=(
            [pltpu.VMEM((_NCHUNK, _C), jnp.int32)]
            + [pltpu.VMEM((_C, CHARSET), jnp.float32) for _ in range(_NBUF)]
            + [pltpu.SemaphoreType.DMA for _ in range(2 * _NBUF)]
        ),
        compiler_params=pltpu.CompilerParams(use_tc_tiling_on_sc=False),
    )
    return f(table, idx3)


def kernel(index, targets, embedding_table):
    idx3 = index.astype(jnp.int32).reshape(_NW, _NCHUNK, _C)
    out = _run(embedding_table, idx3)
    return out.reshape(B, L, CHARSET)
